# baseline (speedup 1.0000x reference)
import functools

import jax
import jax.numpy as jnp
from jax import lax
from jax.experimental import pallas as pl
from jax.experimental.pallas import tpu as pltpu
from jax.experimental.pallas import tpu_sc as plsc

F32 = jnp.float32
I32 = jnp.int32

_D = 64
_L = 16
_CH = 128
_NI = 50000
_NO = 10000
_NU = 10000

_HALF = 25000
_HALFP = 25088
_ZROWS_II = _HALFP // 16

_BLK = 1024


def _l2n(x):
    n = jnp.sqrt(jnp.sum(x * x, axis=-1, keepdims=True))
    return x / jnp.maximum(n, 1e-12)


def _ln_relu(h, g, be):
    mu = jnp.mean(h, axis=-1, keepdims=True)
    var = jnp.mean((h - mu) ** 2, axis=-1, keepdims=True)
    h = (h - mu) / jnp.sqrt(var + 1e-5) * g + be
    return jnp.maximum(h, 0.0)


def _dot(a, b):
    return jnp.dot(a, b, preferred_element_type=F32)


def _fuse_body(img, txt, cat, wi, bi, wt, bt, wc, bc,
               w1i, w1t, w1c, b1, g, be, w2, b2, o):
    xi = _l2n(_dot(img[...], wi[...]) + bi[...])
    xt = _l2n(_dot(txt[...], wt[...]) + bt[...])
    xc = _l2n(_dot(cat[...], wc[...]) + bc[...])
    h = (_dot(xi, w1i[...]) + _dot(xt, w1t[...]) + _dot(xc, w1c[...])
         + b1[...])
    h = _ln_relu(h, g[...], be[...])
    o[...] = _l2n(_dot(h, w2[...]) + b2[...])


def _proj_fuse(image_x, text_x, cat_x, p):
    grid = (pl.cdiv(_NI, _BLK),)

    def row_spec(cols):
        return pl.BlockSpec((_BLK, cols), lambda i: (i, 0))

    def full_spec(shape):
        return pl.BlockSpec(shape, lambda i: (0,) * len(shape))

    w1 = p['fuse_W1']
    args = (image_x, text_x, cat_x,
            p['img_W'], p['img_b'].reshape(1, _D),
            p['txt_W'], p['txt_b'].reshape(1, _D),
            p['cat_W'], p['cat_b'].reshape(1, _D),
            w1[:_D], w1[_D:2 * _D], w1[2 * _D:],
            p['fuse_b1'].reshape(1, _D), p['fuse_g'].reshape(1, _D),
            p['fuse_be'].reshape(1, _D), p['fuse_W2'],
            p['fuse_b2'].reshape(1, _D))
    in_specs = [row_spec(512), row_spec(384), row_spec(128)] + [
        full_spec(a.shape) for a in args[3:]]
    return pl.pallas_call(
        _fuse_body,
        grid=grid,
        in_specs=in_specs,
        out_specs=pl.BlockSpec((_BLK, _D), lambda i: (i, 0)),
        out_shape=jax.ShapeDtypeStruct((_NI, _D), F32),
    )(*args)


def _item_body(x, xp, w1, b1, g, be, w2, b2, o):
    h = _ln_relu(_dot(xp[...], w1[...]) + b1[...], g[...], be[...])
    o[...] = _l2n(x[...] + _dot(h, w2[...]) + b2[...])


def _item_update(x, x_prop, p):
    grid = (pl.cdiv(_NI, _BLK),)
    args = (x, x_prop, p['iu_W1'], p['iu_b1'].reshape(1, _D),
            p['iu_g'].reshape(1, _D), p['iu_be'].reshape(1, _D),
            p['iu_W2'], p['iu_b2'].reshape(1, _D))
    in_specs = [pl.BlockSpec((_BLK, _D), lambda i: (i, 0))] * 2 + [
        pl.BlockSpec(a.shape, lambda i: (0, 0)) for a in args[2:]]
    return pl.pallas_call(
        _item_body,
        grid=grid,
        in_specs=in_specs,
        out_specs=pl.BlockSpec((_BLK, _D), lambda i: (i, 0)),
        out_shape=jax.ShapeDtypeStruct((_NI, _D), F32),
    )(*args)


def _node_body(a0, a1, base, w1, b1, g, be, w2, b2, o):
    agg = a0[...] + a1[...]
    h = _ln_relu(_dot(agg, w1[...]) + b1[...], g[...], be[...])
    o[...] = _l2n(_l2n(base[...]) + _dot(h, w2[...]) + b2[...])


def _node_update(a0, a1, base, p, name):
    n = base.shape[0]
    grid = (pl.cdiv(n, _BLK),)
    args = (a0, a1, base, p[name + '_W1'], p[name + '_b1'].reshape(1, _D),
            p[name + '_g'].reshape(1, _D), p[name + '_be'].reshape(1, _D),
            p[name + '_W2'], p[name + '_b2'].reshape(1, _D))
    in_specs = [pl.BlockSpec((_BLK, _D), lambda i: (i, 0))] * 3 + [
        pl.BlockSpec(a.shape, lambda i: (0, 0)) for a in args[3:]]
    return pl.pallas_call(
        _node_body,
        grid=grid,
        in_specs=in_specs,
        out_specs=pl.BlockSpec((_BLK, _D), lambda i: (i, 0)),
        out_shape=jax.ShapeDtypeStruct((n, _D), F32),
    )(*args)


_SB = 8


def _scale_chunk(valq, buf, voff):
    @plsc.parallel_loop(0, _CH // _L)
    def grp(g):
        v16 = valq[pl.ds(voff + g * _L, _L)]
        for e in range(_L):
            vb = v16.at[jnp.full((_L,), e, I32)].get(mode="promise_in_bounds")
            row = g * _L + e
            for j in range(_D // _L):
                sl = pl.ds(j * _L, _L)
                buf[row, sl] = buf[row, sl] * vb


def _process_super(x_h, acc, colq, idxq, valq, gbufs, gsems, ssems,
                   base_row, transform):
    transform(base_row)
    cps = [None] * 2
    scs = [None] * 2
    cps[0] = pltpu.async_copy(x_h.at[colq.at[base_row]], gbufs[0], gsems[0])
    for b in range(_SB):
        if b + 1 < _SB:
            k = (b + 1) % 2
            if b >= 1:
                scs[k].wait()
            cps[k] = pltpu.async_copy(x_h.at[colq.at[base_row + b + 1]],
                                      gbufs[k], gsems[k])
        cps[b % 2].wait()
        _scale_chunk(valq, gbufs[b % 2], (base_row + b) * _CH)
        scs[b % 2] = pltpu.async_copy(gbufs[b % 2],
                                      acc.at[idxq.at[base_row + b]],
                                      ssems[b % 2], add=True)
    scs[(_SB - 2) % 2].wait()
    scs[(_SB - 1) % 2].wait()


def _process_group(x_h, acc, colq, idxq, valq, gbufs, gsems, ssems,
                   n_super, transform):
    def sup(u, carry):
        _process_super(x_h, acc, colq, idxq, valq, gbufs, gsems, ssems,
                       u * _SB, transform)
        return carry
    lax.fori_loop(0, n_super, sup, 0)


def _stage(colm, rowm, valf, colq, idxq, valq, sems, row0, q_rows):
    pltpu.async_copy(colm.at[pl.ds(row0, q_rows)], colq, sems[0])
    pltpu.async_copy(rowm.at[pl.ds(row0, q_rows)], idxq, sems[1])
    pltpu.async_copy(valf.at[pl.ds(row0 * _CH, q_rows * _CH)],
                     valq.at[pl.ds(0, q_rows * _CH)], sems[2])


def _stage_wait(colm, rowm, valf, colq, idxq, valq, sems, row0, q_rows):
    pltpu.make_async_copy(colm.at[pl.ds(row0, q_rows)], colq, sems[0]).wait()
    pltpu.make_async_copy(rowm.at[pl.ds(row0, q_rows)], idxq, sems[1]).wait()
    pltpu.make_async_copy(valf.at[pl.ds(row0 * _CH, q_rows * _CH)],
                          valq.at[pl.ds(0, q_rows * _CH)], sems[2]).wait()


def _sc_scratch(n_acc_rows, q_rows, double):
    n_sets = 2 if double else 1
    sc = []
    for _ in range(n_sets):
        sc += [
            pltpu.VMEM((q_rows, _CH), I32),
            pltpu.VMEM((q_rows, _CH), I32),
            pltpu.VMEM((q_rows * _CH + _L,), F32),
            (pltpu.SemaphoreType.DMA,) * 3,
        ]
    sc += [
        tuple(pltpu.VMEM((_CH, _D), F32) for _ in range(2)),
        (pltpu.SemaphoreType.DMA,) * 2,
        (pltpu.SemaphoreType.DMA,) * 2,
        pltpu.VMEM_SHARED((n_acc_rows, _D), F32),
    ]
    return sc


def _spmm_ii(rows, cols, vals, x, zeros, n_groups, q_super):
    e_pad = rows.shape[0]
    per_tile_rows = e_pad // _CH // 16
    q_rows = q_super * _SB
    n_pairs = n_groups // 2
    rowm = rows.reshape(-1, _CH)
    colm = cols.reshape(-1, _CH)
    mesh = plsc.VectorSubcoreMesh(core_axis_name="c", subcore_axis_name="s")

    @functools.partial(
        pl.kernel,
        out_type=jax.ShapeDtypeStruct((2 * _HALFP, _D), F32),
        mesh=mesh,
        scratch_types=_sc_scratch(_HALFP, q_rows, True),
        compiler_params=pltpu.CompilerParams(use_tc_tiling_on_sc=False),
    )
    def k(row_h, col_h, val_h, x_h, z_h, out_h,
          col_a, idx_a, val_a, st_a, col_b, idx_b, val_b, st_b,
          gbufs, gsems, ssems, acc):
        c = lax.axis_index("c")
        s = lax.axis_index("s")
        zb = s * _ZROWS_II
        pltpu.sync_copy(z_h.at[pl.ds(zb, _ZROWS_II)],
                        acc.at[pl.ds(zb, _ZROWS_II)])
        plsc.subcore_barrier()

        def make_transform(idxq, valq):
            def transform(base_row):
                for b in range(_SB):
                    for g in range(_CH // _L):
                        sl = pl.ds(g * _L, _L)
                        r = idxq[base_row + b, sl]
                        hi = r >= _HALF
                        idxq[base_row + b, sl] = jnp.where(hi, r - _HALF, r)
                        side = jnp.where(hi, jnp.ones((_L,), I32),
                                         jnp.zeros((_L,), I32))
                        vsl = pl.ds((base_row + b) * _CH + g * _L, _L)
                        valq[vsl] = jnp.where(side == c, valq[vsl],
                                              jnp.zeros((_L,), F32))
            return transform

        tile_row0 = s * per_tile_rows
        set_a = (col_a, idx_a, val_a, st_a)
        set_b = (col_b, idx_b, val_b, st_b)

        def stage_for(st, row0):
            _stage(col_h, row_h, val_h, st[0], st[1], st[2], st[3],
                   row0, q_rows)

        def wait_for(st, row0):
            _stage_wait(col_h, row_h, val_h, st[0], st[1], st[2], st[3],
                        row0, q_rows)

        def process(st):
            _process_group(x_h, acc, st[0], st[1], st[2], gbufs, gsems,
                           ssems, q_super, make_transform(st[1], st[2]))

        stage_for(set_a, tile_row0)

        def pair(q, carry):
            ra = tile_row0 + (2 * q) * q_rows
            rb = ra + q_rows
            wait_for(set_a, ra)
            stage_for(set_b, rb)
            process(set_a)

            @pl.when(q + 1 < n_pairs)
            def _():
                stage_for(set_a, ra + 2 * q_rows)

            wait_for(set_b, rb)
            process(set_b)
            return carry

        lax.fori_loop(0, n_pairs, pair, 0)
        plsc.subcore_barrier()
        pltpu.sync_copy(acc.at[pl.ds(zb, _ZROWS_II)],
                        out_h.at[pl.ds(c * _HALFP + zb, _ZROWS_II)])

    return k(rowm, colm, vals, x, zeros)


def _spmm_part(rows, cols, vals, x, zeros, n_dst):
    n_pad = 128 * pl.cdiv(n_dst, 128)
    per_rows = n_pad // 16
    e_pad = rows.shape[0]
    per_tile_rows = e_pad // _CH // 32
    q_super = per_tile_rows // _SB
    q_rows = per_tile_rows
    rowm = rows.reshape(-1, _CH)
    colm = cols.reshape(-1, _CH)
    zeros = zeros[:n_pad]
    mesh = plsc.VectorSubcoreMesh(core_axis_name="c", subcore_axis_name="s")

    @functools.partial(
        pl.kernel,
        out_type=jax.ShapeDtypeStruct((2, n_pad, _D), F32),
        mesh=mesh,
        scratch_types=_sc_scratch(n_pad, q_rows, False),
        compiler_params=pltpu.CompilerParams(use_tc_tiling_on_sc=False),
    )
    def k(row_h, col_h, val_h, x_h, z_h, out_h,
          col_a, idx_a, val_a, st_a, gbufs, gsems, ssems, acc):
        c = lax.axis_index("c")
        s = lax.axis_index("s")
        zb = s * per_rows
        pltpu.sync_copy(z_h.at[pl.ds(zb, per_rows)],
                        acc.at[pl.ds(zb, per_rows)])
        plsc.subcore_barrier()

        row0 = (s * 2 + c) * per_tile_rows
        _stage(col_h, row_h, val_h, col_a, idx_a, val_a, st_a, row0, q_rows)
        _stage_wait(col_h, row_h, val_h, col_a, idx_a, val_a, st_a,
                    row0, q_rows)
        _process_group(x_h, acc, col_a, idx_a, val_a, gbufs, gsems, ssems,
                       q_super, lambda base_row: None)
        plsc.subcore_barrier()
        pltpu.sync_copy(acc.at[pl.ds(zb, per_rows)],
                        out_h.at[c, pl.ds(zb, per_rows)])

    return k(rowm, colm, vals, x, zeros)


def _pad_edges(r, c, v, e_pad, n_dst, n_src):
    pad = e_pad - r.shape[0]
    spread = jnp.arange(pad, dtype=I32)
    r = jnp.concatenate([r.astype(I32), spread % n_dst])
    c = jnp.concatenate([c.astype(I32), spread % n_src])
    v = jnp.concatenate([v, jnp.zeros((pad,), F32)])
    return r, c, v


def kernel(image_x, text_x, cat_x, A_ii_idx, A_ii_val, A_oi_row, A_oi_col,
           A_oi_val, A_uo_row, A_uo_col, A_uo_val, params):
    p = params
    zeros = jnp.zeros((_HALFP, _D), F32)

    x = _proj_fuse(image_x, text_x, cat_x, p)

    ii_groups, ii_qsuper = 26, 2
    ii_gran = 16 * ii_groups * ii_qsuper * _SB * _CH
    e_pad = ii_gran * pl.cdiv(A_ii_val.shape[0], ii_gran)
    r, c, v = _pad_edges(A_ii_idx[0], A_ii_idx[1], A_ii_val, e_pad, _NI, _NI)
    xp = _spmm_ii(r, c, v, x, zeros, ii_groups, ii_qsuper)
    x_prop = xp.reshape(2, _HALFP, _D)[:, :_HALF].reshape(_NI, _D)

    item_emb = _item_update(x, x_prop, p)

    e_pad = 32 * _SB * _CH * pl.cdiv(A_oi_val.shape[0], 32 * _SB * _CH)
    r, c, v = _pad_edges(A_oi_row, A_oi_col, A_oi_val, e_pad, _NO, _NI)
    agg = _spmm_part(r, c, v, item_emb, zeros, _NO)
    outfit_emb = _node_update(agg[0, :_NO], agg[1, :_NO],
                              p['outfit_base'], p, 'ou')

    e_pad = 32 * _SB * _CH * pl.cdiv(A_uo_val.shape[0], 32 * _SB * _CH)
    r, c, v = _pad_edges(A_uo_row, A_uo_col, A_uo_val, e_pad, _NU, _NO)
    agg2 = _spmm_part(r, c, v, outfit_emb, zeros, _NU)
    user_emb = _node_update(agg2[0, :_NU], agg2[1, :_NU],
                            p['user_base'], p, 'uu')

    return (user_emb, outfit_emb, item_emb)


# speedup vs baseline: 1.0549x; 1.0549x over previous
import functools

import jax
import jax.numpy as jnp
from jax import lax
from jax.experimental import pallas as pl
from jax.experimental.pallas import tpu as pltpu
from jax.experimental.pallas import tpu_sc as plsc

F32 = jnp.float32
I32 = jnp.int32

_D = 64
_L = 16
_CH = 128
_NI = 50000
_NO = 10000
_NU = 10000

_HALF = 25000
_HALFP = 25088
_ZROWS_II = _HALFP // 16

_BLK = 1024


def _l2n(x):
    n = jnp.sqrt(jnp.sum(x * x, axis=-1, keepdims=True))
    return x / jnp.maximum(n, 1e-12)


def _ln_relu(h, g, be):
    mu = jnp.mean(h, axis=-1, keepdims=True)
    var = jnp.mean((h - mu) ** 2, axis=-1, keepdims=True)
    h = (h - mu) / jnp.sqrt(var + 1e-5) * g + be
    return jnp.maximum(h, 0.0)


def _dot(a, b):
    return jnp.dot(a, b, preferred_element_type=F32)


def _fuse_body(img, txt, cat, wi, bi, wt, bt, wc, bc,
               w1i, w1t, w1c, b1, g, be, w2, b2, o):
    xi = _l2n(_dot(img[...], wi[...]) + bi[...])
    xt = _l2n(_dot(txt[...], wt[...]) + bt[...])
    xc = _l2n(_dot(cat[...], wc[...]) + bc[...])
    h = (_dot(xi, w1i[...]) + _dot(xt, w1t[...]) + _dot(xc, w1c[...])
         + b1[...])
    h = _ln_relu(h, g[...], be[...])
    o[...] = _l2n(_dot(h, w2[...]) + b2[...])


def _proj_fuse(image_x, text_x, cat_x, p):
    grid = (pl.cdiv(_NI, _BLK),)

    def row_spec(cols):
        return pl.BlockSpec((_BLK, cols), lambda i: (i, 0))

    def full_spec(shape):
        return pl.BlockSpec(shape, lambda i: (0,) * len(shape))

    w1 = p['fuse_W1']
    args = (image_x, text_x, cat_x,
            p['img_W'], p['img_b'].reshape(1, _D),
            p['txt_W'], p['txt_b'].reshape(1, _D),
            p['cat_W'], p['cat_b'].reshape(1, _D),
            w1[:_D], w1[_D:2 * _D], w1[2 * _D:],
            p['fuse_b1'].reshape(1, _D), p['fuse_g'].reshape(1, _D),
            p['fuse_be'].reshape(1, _D), p['fuse_W2'],
            p['fuse_b2'].reshape(1, _D))
    in_specs = [row_spec(512), row_spec(384), row_spec(128)] + [
        full_spec(a.shape) for a in args[3:]]
    return pl.pallas_call(
        _fuse_body,
        grid=grid,
        in_specs=in_specs,
        out_specs=pl.BlockSpec((_BLK, _D), lambda i: (i, 0)),
        out_shape=jax.ShapeDtypeStruct((_NI, _D), F32),
    )(*args)


def _item_body(x, xp, w1, b1, g, be, w2, b2, o):
    h = _ln_relu(_dot(xp[...], w1[...]) + b1[...], g[...], be[...])
    o[...] = _l2n(x[...] + _dot(h, w2[...]) + b2[...])


def _item_update(x, x_prop, p):
    grid = (pl.cdiv(_NI, _BLK),)
    args = (x, x_prop, p['iu_W1'], p['iu_b1'].reshape(1, _D),
            p['iu_g'].reshape(1, _D), p['iu_be'].reshape(1, _D),
            p['iu_W2'], p['iu_b2'].reshape(1, _D))
    in_specs = [pl.BlockSpec((_BLK, _D), lambda i: (i, 0))] * 2 + [
        pl.BlockSpec(a.shape, lambda i: (0, 0)) for a in args[2:]]
    return pl.pallas_call(
        _item_body,
        grid=grid,
        in_specs=in_specs,
        out_specs=pl.BlockSpec((_BLK, _D), lambda i: (i, 0)),
        out_shape=jax.ShapeDtypeStruct((_NI, _D), F32),
    )(*args)


def _node_body(a0, a1, base, w1, b1, g, be, w2, b2, o):
    agg = a0[...] + a1[...]
    h = _ln_relu(_dot(agg, w1[...]) + b1[...], g[...], be[...])
    o[...] = _l2n(_l2n(base[...]) + _dot(h, w2[...]) + b2[...])


def _node_update(a0, a1, base, p, name):
    n = base.shape[0]
    grid = (pl.cdiv(n, _BLK),)
    args = (a0, a1, base, p[name + '_W1'], p[name + '_b1'].reshape(1, _D),
            p[name + '_g'].reshape(1, _D), p[name + '_be'].reshape(1, _D),
            p[name + '_W2'], p[name + '_b2'].reshape(1, _D))
    in_specs = [pl.BlockSpec((_BLK, _D), lambda i: (i, 0))] * 3 + [
        pl.BlockSpec(a.shape, lambda i: (0, 0)) for a in args[3:]]
    return pl.pallas_call(
        _node_body,
        grid=grid,
        in_specs=in_specs,
        out_specs=pl.BlockSpec((_BLK, _D), lambda i: (i, 0)),
        out_shape=jax.ShapeDtypeStruct((n, _D), F32),
    )(*args)


_SB = 8


def _scale_chunk(valq, buf, voff):
    @plsc.parallel_loop(0, _CH, unroll=8)
    def edge(i):
        vb = jnp.full((_L,), valq[pl.ds(voff + i, _L)][0], F32)
        for j in range(_D // _L):
            sl = pl.ds(j * _L, _L)
            buf[i, sl] = buf[i, sl] * vb


def _process_super(x_h, acc, colq, idxq, valq, gbufs, gsems, ssems,
                   base_row, transform):
    transform(base_row)
    cps = [None] * 2
    scs = [None] * 2
    cps[0] = pltpu.async_copy(x_h.at[colq.at[base_row]], gbufs[0], gsems[0])
    for b in range(_SB):
        if b + 1 < _SB:
            k = (b + 1) % 2
            if b >= 1:
                scs[k].wait()
            cps[k] = pltpu.async_copy(x_h.at[colq.at[base_row + b + 1]],
                                      gbufs[k], gsems[k])
        cps[b % 2].wait()
        _scale_chunk(valq, gbufs[b % 2], (base_row + b) * _CH)
        scs[b % 2] = pltpu.async_copy(gbufs[b % 2],
                                      acc.at[idxq.at[base_row + b]],
                                      ssems[b % 2], add=True)
    scs[(_SB - 2) % 2].wait()
    scs[(_SB - 1) % 2].wait()


def _process_group(x_h, acc, colq, idxq, valq, gbufs, gsems, ssems,
                   n_super, transform):
    def sup(u, carry):
        _process_super(x_h, acc, colq, idxq, valq, gbufs, gsems, ssems,
                       u * _SB, transform)
        return carry
    lax.fori_loop(0, n_super, sup, 0)


def _stage(colm, rowm, valf, colq, idxq, valq, sems, row0, q_rows):
    pltpu.async_copy(colm.at[pl.ds(row0, q_rows)], colq, sems[0])
    pltpu.async_copy(rowm.at[pl.ds(row0, q_rows)], idxq, sems[1])
    pltpu.async_copy(valf.at[pl.ds(row0 * _CH, q_rows * _CH)],
                     valq.at[pl.ds(0, q_rows * _CH)], sems[2])


def _stage_wait(colm, rowm, valf, colq, idxq, valq, sems, row0, q_rows):
    pltpu.make_async_copy(colm.at[pl.ds(row0, q_rows)], colq, sems[0]).wait()
    pltpu.make_async_copy(rowm.at[pl.ds(row0, q_rows)], idxq, sems[1]).wait()
    pltpu.make_async_copy(valf.at[pl.ds(row0 * _CH, q_rows * _CH)],
                          valq.at[pl.ds(0, q_rows * _CH)], sems[2]).wait()


def _sc_scratch(n_acc_rows, q_rows, double):
    n_sets = 2 if double else 1
    sc = []
    for _ in range(n_sets):
        sc += [
            pltpu.VMEM((q_rows, _CH), I32),
            pltpu.VMEM((q_rows, _CH), I32),
            pltpu.VMEM((q_rows * _CH + _L,), F32),
            (pltpu.SemaphoreType.DMA,) * 3,
        ]
    sc += [
        tuple(pltpu.VMEM((_CH, _D), F32) for _ in range(2)),
        (pltpu.SemaphoreType.DMA,) * 2,
        (pltpu.SemaphoreType.DMA,) * 2,
        pltpu.VMEM_SHARED((n_acc_rows, _D), F32),
    ]
    return sc


def _spmm_ii(rows, cols, vals, x, zeros, n_groups, q_super):
    e_pad = rows.shape[0]
    per_tile_rows = e_pad // _CH // 16
    q_rows = q_super * _SB
    n_pairs = n_groups // 2
    rowm = rows.reshape(-1, _CH)
    colm = cols.reshape(-1, _CH)
    mesh = plsc.VectorSubcoreMesh(core_axis_name="c", subcore_axis_name="s")

    @functools.partial(
        pl.kernel,
        out_type=jax.ShapeDtypeStruct((2 * _HALFP, _D), F32),
        mesh=mesh,
        scratch_types=_sc_scratch(_HALFP, q_rows, True),
        compiler_params=pltpu.CompilerParams(use_tc_tiling_on_sc=False),
    )
    def k(row_h, col_h, val_h, x_h, z_h, out_h,
          col_a, idx_a, val_a, st_a, col_b, idx_b, val_b, st_b,
          gbufs, gsems, ssems, acc):
        c = lax.axis_index("c")
        s = lax.axis_index("s")
        zb = s * _ZROWS_II
        pltpu.sync_copy(z_h.at[pl.ds(zb, _ZROWS_II)],
                        acc.at[pl.ds(zb, _ZROWS_II)])
        plsc.subcore_barrier()

        def make_transform(idxq, valq):
            def transform(base_row):
                for b in range(_SB):
                    for g in range(_CH // _L):
                        sl = pl.ds(g * _L, _L)
                        r = idxq[base_row + b, sl]
                        hi = r >= _HALF
                        idxq[base_row + b, sl] = jnp.where(hi, r - _HALF, r)
                        side = jnp.where(hi, jnp.ones((_L,), I32),
                                         jnp.zeros((_L,), I32))
                        vsl = pl.ds((base_row + b) * _CH + g * _L, _L)
                        valq[vsl] = jnp.where(side == c, valq[vsl],
                                              jnp.zeros((_L,), F32))
            return transform

        tile_row0 = s * per_tile_rows
        set_a = (col_a, idx_a, val_a, st_a)
        set_b = (col_b, idx_b, val_b, st_b)

        def stage_for(st, row0):
            _stage(col_h, row_h, val_h, st[0], st[1], st[2], st[3],
                   row0, q_rows)

        def wait_for(st, row0):
            _stage_wait(col_h, row_h, val_h, st[0], st[1], st[2], st[3],
                        row0, q_rows)

        def process(st):
            _process_group(x_h, acc, st[0], st[1], st[2], gbufs, gsems,
                           ssems, q_super, make_transform(st[1], st[2]))

        stage_for(set_a, tile_row0)

        def pair(q, carry):
            ra = tile_row0 + (2 * q) * q_rows
            rb = ra + q_rows
            wait_for(set_a, ra)
            stage_for(set_b, rb)
            process(set_a)

            @pl.when(q + 1 < n_pairs)
            def _():
                stage_for(set_a, ra + 2 * q_rows)

            wait_for(set_b, rb)
            process(set_b)
            return carry

        lax.fori_loop(0, n_pairs, pair, 0)
        plsc.subcore_barrier()
        pltpu.sync_copy(acc.at[pl.ds(zb, _ZROWS_II)],
                        out_h.at[pl.ds(c * _HALFP + zb, _ZROWS_II)])

    return k(rowm, colm, vals, x, zeros)


def _spmm_part(rows, cols, vals, x, zeros, n_dst):
    n_pad = 128 * pl.cdiv(n_dst, 128)
    per_rows = n_pad // 16
    e_pad = rows.shape[0]
    per_tile_rows = e_pad // _CH // 32
    q_super = per_tile_rows // _SB
    q_rows = per_tile_rows
    rowm = rows.reshape(-1, _CH)
    colm = cols.reshape(-1, _CH)
    zeros = zeros[:n_pad]
    mesh = plsc.VectorSubcoreMesh(core_axis_name="c", subcore_axis_name="s")

    @functools.partial(
        pl.kernel,
        out_type=jax.ShapeDtypeStruct((2, n_pad, _D), F32),
        mesh=mesh,
        scratch_types=_sc_scratch(n_pad, q_rows, False),
        compiler_params=pltpu.CompilerParams(use_tc_tiling_on_sc=False),
    )
    def k(row_h, col_h, val_h, x_h, z_h, out_h,
          col_a, idx_a, val_a, st_a, gbufs, gsems, ssems, acc):
        c = lax.axis_index("c")
        s = lax.axis_index("s")
        zb = s * per_rows
        pltpu.sync_copy(z_h.at[pl.ds(zb, per_rows)],
                        acc.at[pl.ds(zb, per_rows)])
        plsc.subcore_barrier()

        row0 = (s * 2 + c) * per_tile_rows
        _stage(col_h, row_h, val_h, col_a, idx_a, val_a, st_a, row0, q_rows)
        _stage_wait(col_h, row_h, val_h, col_a, idx_a, val_a, st_a,
                    row0, q_rows)
        _process_group(x_h, acc, col_a, idx_a, val_a, gbufs, gsems, ssems,
                       q_super, lambda base_row: None)
        plsc.subcore_barrier()
        pltpu.sync_copy(acc.at[pl.ds(zb, per_rows)],
                        out_h.at[c, pl.ds(zb, per_rows)])

    return k(rowm, colm, vals, x, zeros)


def _pad_edges(r, c, v, e_pad, n_dst, n_src):
    pad = e_pad - r.shape[0]
    spread = jnp.arange(pad, dtype=I32)
    r = jnp.concatenate([r.astype(I32), spread % n_dst])
    c = jnp.concatenate([c.astype(I32), spread % n_src])
    v = jnp.concatenate([v, jnp.zeros((pad,), F32)])
    return r, c, v


def kernel(image_x, text_x, cat_x, A_ii_idx, A_ii_val, A_oi_row, A_oi_col,
           A_oi_val, A_uo_row, A_uo_col, A_uo_val, params):
    p = params
    zeros = jnp.zeros((_HALFP, _D), F32)

    x = _proj_fuse(image_x, text_x, cat_x, p)

    ii_groups, ii_qsuper = 26, 2
    ii_gran = 16 * ii_groups * ii_qsuper * _SB * _CH
    e_pad = ii_gran * pl.cdiv(A_ii_val.shape[0], ii_gran)
    r, c, v = _pad_edges(A_ii_idx[0], A_ii_idx[1], A_ii_val, e_pad, _NI, _NI)
    xp = _spmm_ii(r, c, v, x, zeros, ii_groups, ii_qsuper)
    x_prop = xp.reshape(2, _HALFP, _D)[:, :_HALF].reshape(_NI, _D)

    item_emb = _item_update(x, x_prop, p)

    e_pad = 32 * _SB * _CH * pl.cdiv(A_oi_val.shape[0], 32 * _SB * _CH)
    r, c, v = _pad_edges(A_oi_row, A_oi_col, A_oi_val, e_pad, _NO, _NI)
    agg = _spmm_part(r, c, v, item_emb, zeros, _NO)
    outfit_emb = _node_update(agg[0, :_NO], agg[1, :_NO],
                              p['outfit_base'], p, 'ou')

    e_pad = 32 * _SB * _CH * pl.cdiv(A_uo_val.shape[0], 32 * _SB * _CH)
    r, c, v = _pad_edges(A_uo_row, A_uo_col, A_uo_val, e_pad, _NU, _NO)
    agg2 = _spmm_part(r, c, v, outfit_emb, zeros, _NU)
    user_emb = _node_update(agg2[0, :_NU], agg2[1, :_NU],
                            p['user_base'], p, 'uu')

    return (user_emb, outfit_emb, item_emb)


# speedup vs baseline: 1.0740x; 1.0180x over previous
import functools

import jax
import jax.numpy as jnp
from jax import lax
from jax.experimental import pallas as pl
from jax.experimental.pallas import tpu as pltpu
from jax.experimental.pallas import tpu_sc as plsc

F32 = jnp.float32
I32 = jnp.int32

_D = 64
_L = 16
_CH = 128
_NI = 50000
_NO = 10000
_NU = 10000

_HALF = 25000
_HALFP = 25088
_ZROWS_II = _HALFP // 16

_BLK = 1024


def _l2n(x):
    n = jnp.sqrt(jnp.sum(x * x, axis=-1, keepdims=True))
    return x / jnp.maximum(n, 1e-12)


def _ln_relu(h, g, be):
    mu = jnp.mean(h, axis=-1, keepdims=True)
    var = jnp.mean((h - mu) ** 2, axis=-1, keepdims=True)
    h = (h - mu) / jnp.sqrt(var + 1e-5) * g + be
    return jnp.maximum(h, 0.0)


def _dot(a, b):
    return jnp.dot(a, b, preferred_element_type=F32)


def _fuse_body(img, txt, cat, wi, bi, wt, bt, wc, bc,
               w1i, w1t, w1c, b1, g, be, w2, b2, o):
    xi = _l2n(_dot(img[...], wi[...]) + bi[...])
    xt = _l2n(_dot(txt[...], wt[...]) + bt[...])
    xc = _l2n(_dot(cat[...], wc[...]) + bc[...])
    h = (_dot(xi, w1i[...]) + _dot(xt, w1t[...]) + _dot(xc, w1c[...])
         + b1[...])
    h = _ln_relu(h, g[...], be[...])
    o[...] = _l2n(_dot(h, w2[...]) + b2[...])


def _proj_fuse(image_x, text_x, cat_x, p):
    grid = (pl.cdiv(_NI, _BLK),)

    def row_spec(cols):
        return pl.BlockSpec((_BLK, cols), lambda i: (i, 0))

    def full_spec(shape):
        return pl.BlockSpec(shape, lambda i: (0,) * len(shape))

    w1 = p['fuse_W1']
    args = (image_x, text_x, cat_x,
            p['img_W'], p['img_b'].reshape(1, _D),
            p['txt_W'], p['txt_b'].reshape(1, _D),
            p['cat_W'], p['cat_b'].reshape(1, _D),
            w1[:_D], w1[_D:2 * _D], w1[2 * _D:],
            p['fuse_b1'].reshape(1, _D), p['fuse_g'].reshape(1, _D),
            p['fuse_be'].reshape(1, _D), p['fuse_W2'],
            p['fuse_b2'].reshape(1, _D))
    in_specs = [row_spec(512), row_spec(384), row_spec(128)] + [
        full_spec(a.shape) for a in args[3:]]
    return pl.pallas_call(
        _fuse_body,
        grid=grid,
        in_specs=in_specs,
        out_specs=pl.BlockSpec((_BLK, _D), lambda i: (i, 0)),
        out_shape=jax.ShapeDtypeStruct((_NI, _D), F32),
    )(*args)


def _item_body(x, xp, w1, b1, g, be, w2, b2, o):
    h = _ln_relu(_dot(xp[...], w1[...]) + b1[...], g[...], be[...])
    o[...] = _l2n(x[...] + _dot(h, w2[...]) + b2[...])


def _item_update(x, x_prop, p):
    grid = (pl.cdiv(_NI, _BLK),)
    args = (x, x_prop, p['iu_W1'], p['iu_b1'].reshape(1, _D),
            p['iu_g'].reshape(1, _D), p['iu_be'].reshape(1, _D),
            p['iu_W2'], p['iu_b2'].reshape(1, _D))
    in_specs = [pl.BlockSpec((_BLK, _D), lambda i: (i, 0))] * 2 + [
        pl.BlockSpec(a.shape, lambda i: (0, 0)) for a in args[2:]]
    return pl.pallas_call(
        _item_body,
        grid=grid,
        in_specs=in_specs,
        out_specs=pl.BlockSpec((_BLK, _D), lambda i: (i, 0)),
        out_shape=jax.ShapeDtypeStruct((_NI, _D), F32),
    )(*args)


def _node_body(a0, a1, base, w1, b1, g, be, w2, b2, o):
    agg = a0[...] + a1[...]
    h = _ln_relu(_dot(agg, w1[...]) + b1[...], g[...], be[...])
    o[...] = _l2n(_l2n(base[...]) + _dot(h, w2[...]) + b2[...])


def _node_update(a0, a1, base, p, name):
    n = base.shape[0]
    grid = (pl.cdiv(n, _BLK),)
    args = (a0, a1, base, p[name + '_W1'], p[name + '_b1'].reshape(1, _D),
            p[name + '_g'].reshape(1, _D), p[name + '_be'].reshape(1, _D),
            p[name + '_W2'], p[name + '_b2'].reshape(1, _D))
    in_specs = [pl.BlockSpec((_BLK, _D), lambda i: (i, 0))] * 3 + [
        pl.BlockSpec(a.shape, lambda i: (0, 0)) for a in args[3:]]
    return pl.pallas_call(
        _node_body,
        grid=grid,
        in_specs=in_specs,
        out_specs=pl.BlockSpec((_BLK, _D), lambda i: (i, 0)),
        out_shape=jax.ShapeDtypeStruct((n, _D), F32),
    )(*args)


_SB = 8


def _scale_chunk(valq, buf, voff):
    @plsc.parallel_loop(0, _CH, unroll=8)
    def edge(i):
        vb = jnp.full((_L,), valq[pl.ds(voff + i, _L)][0], F32)
        for j in range(_D // _L):
            sl = pl.ds(j * _L, _L)
            buf[i, sl] = buf[i, sl] * vb


def _process_super(x_h, acc, colq, idxq, valq, gbufs, gsems, ssems,
                   base_row, transform):
    transform(base_row)
    cps = [None] * 2
    scs = [None] * 2
    cps[0] = pltpu.async_copy(x_h.at[colq.at[base_row]], gbufs[0], gsems[0])
    for b in range(_SB):
        if b + 1 < _SB:
            k = (b + 1) % 2
            if b >= 1:
                scs[k].wait()
            cps[k] = pltpu.async_copy(x_h.at[colq.at[base_row + b + 1]],
                                      gbufs[k], gsems[k])
        cps[b % 2].wait()
        _scale_chunk(valq, gbufs[b % 2], (base_row + b) * _CH)
        scs[b % 2] = pltpu.async_copy(gbufs[b % 2],
                                      acc.at[idxq.at[base_row + b]],
                                      ssems[b % 2], add=True)
    scs[(_SB - 2) % 2].wait()
    scs[(_SB - 1) % 2].wait()


def _process_group(x_h, acc, colq, idxq, valq, gbufs, gsems, ssems,
                   n_super, transform):
    def sup(u, carry):
        _process_super(x_h, acc, colq, idxq, valq, gbufs, gsems, ssems,
                       u * _SB, transform)
        return carry
    lax.fori_loop(0, n_super, sup, 0)


def _stage(colm, rowm, valf, colq, idxq, valq, sems, row0, q_rows):
    pltpu.async_copy(colm.at[pl.ds(row0, q_rows)], colq, sems[0])
    pltpu.async_copy(rowm.at[pl.ds(row0, q_rows)], idxq, sems[1])
    pltpu.async_copy(valf.at[pl.ds(row0 * _CH, q_rows * _CH)],
                     valq.at[pl.ds(0, q_rows * _CH)], sems[2])


def _stage_wait(colm, rowm, valf, colq, idxq, valq, sems, row0, q_rows):
    pltpu.make_async_copy(colm.at[pl.ds(row0, q_rows)], colq, sems[0]).wait()
    pltpu.make_async_copy(rowm.at[pl.ds(row0, q_rows)], idxq, sems[1]).wait()
    pltpu.make_async_copy(valf.at[pl.ds(row0 * _CH, q_rows * _CH)],
                          valq.at[pl.ds(0, q_rows * _CH)], sems[2]).wait()


def _sc_scratch(n_acc_rows, q_rows, double):
    n_sets = 2 if double else 1
    sc = []
    for _ in range(n_sets):
        sc += [
            pltpu.VMEM((q_rows, _CH), I32),
            pltpu.VMEM((q_rows, _CH), I32),
            pltpu.VMEM((q_rows * _CH + _L,), F32),
            (pltpu.SemaphoreType.DMA,) * 3,
        ]
    sc += [
        tuple(pltpu.VMEM((_CH, _D), F32) for _ in range(2)),
        (pltpu.SemaphoreType.DMA,) * 2,
        (pltpu.SemaphoreType.DMA,) * 2,
        pltpu.VMEM_SHARED((n_acc_rows, _D), F32),
    ]
    return sc


def _spmm_ii(rows, cols, vals, x, zeros, n_groups, q_super):
    e_pad = rows.shape[0]
    per_tile_rows = e_pad // _CH // 16
    q_rows = q_super * _SB
    n_pairs = n_groups // 2
    rowm = rows.reshape(-1, _CH)
    colm = cols.reshape(-1, _CH)
    mesh = plsc.VectorSubcoreMesh(core_axis_name="c", subcore_axis_name="s")

    @functools.partial(
        pl.kernel,
        out_type=jax.ShapeDtypeStruct((_NI, _D), F32),
        mesh=mesh,
        scratch_types=_sc_scratch(_HALFP, q_rows, True),
        compiler_params=pltpu.CompilerParams(use_tc_tiling_on_sc=False),
    )
    def k(row_h, col_h, val_h, x_h, z_h, out_h,
          col_a, idx_a, val_a, st_a, col_b, idx_b, val_b, st_b,
          gbufs, gsems, ssems, acc):
        c = lax.axis_index("c")
        s = lax.axis_index("s")
        zb = s * _ZROWS_II
        pltpu.sync_copy(z_h.at[pl.ds(zb, _ZROWS_II)],
                        acc.at[pl.ds(zb, _ZROWS_II)])
        plsc.subcore_barrier()

        def make_transform(idxq, valq):
            def transform(base_row):
                for b in range(_SB):
                    for g in range(_CH // _L):
                        sl = pl.ds(g * _L, _L)
                        r = idxq[base_row + b, sl]
                        hi = r >= _HALF
                        idxq[base_row + b, sl] = jnp.where(hi, r - _HALF, r)
                        side = jnp.where(hi, jnp.ones((_L,), I32),
                                         jnp.zeros((_L,), I32))
                        vsl = pl.ds((base_row + b) * _CH + g * _L, _L)
                        valq[vsl] = jnp.where(side == c, valq[vsl],
                                              jnp.zeros((_L,), F32))
            return transform

        tile_row0 = s * per_tile_rows
        set_a = (col_a, idx_a, val_a, st_a)
        set_b = (col_b, idx_b, val_b, st_b)

        def stage_for(st, row0):
            _stage(col_h, row_h, val_h, st[0], st[1], st[2], st[3],
                   row0, q_rows)

        def wait_for(st, row0):
            _stage_wait(col_h, row_h, val_h, st[0], st[1], st[2], st[3],
                        row0, q_rows)

        def process(st):
            _process_group(x_h, acc, st[0], st[1], st[2], gbufs, gsems,
                           ssems, q_super, make_transform(st[1], st[2]))

        stage_for(set_a, tile_row0)

        def pair(q, carry):
            ra = tile_row0 + (2 * q) * q_rows
            rb = ra + q_rows
            wait_for(set_a, ra)
            stage_for(set_b, rb)
            process(set_a)

            @pl.when(q + 1 < n_pairs)
            def _():
                stage_for(set_a, ra + 2 * q_rows)

            wait_for(set_b, rb)
            process(set_b)
            return carry

        lax.fori_loop(0, n_pairs, pair, 0)
        plsc.subcore_barrier()
        ob = jnp.minimum(s * _ZROWS_II, _HALF - _ZROWS_II)
        pltpu.sync_copy(acc.at[pl.ds(ob, _ZROWS_II)],
                        out_h.at[pl.ds(c * _HALF + ob, _ZROWS_II)])

    return k(rowm, colm, vals, x, zeros)


def _spmm_part(rows, cols, vals, x, zeros, n_dst):
    n_pad = 128 * pl.cdiv(n_dst, 128)
    per_rows = n_pad // 16
    e_pad = rows.shape[0]
    per_tile_rows = e_pad // _CH // 32
    q_super = per_tile_rows // _SB
    q_rows = per_tile_rows
    rowm = rows.reshape(-1, _CH)
    colm = cols.reshape(-1, _CH)
    zeros = zeros[:n_pad]
    mesh = plsc.VectorSubcoreMesh(core_axis_name="c", subcore_axis_name="s")

    @functools.partial(
        pl.kernel,
        out_type=jax.ShapeDtypeStruct((2, n_dst, _D), F32),
        mesh=mesh,
        scratch_types=_sc_scratch(n_pad, q_rows, False),
        compiler_params=pltpu.CompilerParams(use_tc_tiling_on_sc=False),
    )
    def k(row_h, col_h, val_h, x_h, z_h, out_h,
          col_a, idx_a, val_a, st_a, gbufs, gsems, ssems, acc):
        c = lax.axis_index("c")
        s = lax.axis_index("s")
        zb = s * per_rows
        pltpu.sync_copy(z_h.at[pl.ds(zb, per_rows)],
                        acc.at[pl.ds(zb, per_rows)])
        plsc.subcore_barrier()

        row0 = (s * 2 + c) * per_tile_rows
        _stage(col_h, row_h, val_h, col_a, idx_a, val_a, st_a, row0, q_rows)
        _stage_wait(col_h, row_h, val_h, col_a, idx_a, val_a, st_a,
                    row0, q_rows)
        _process_group(x_h, acc, col_a, idx_a, val_a, gbufs, gsems, ssems,
                       q_super, lambda base_row: None)
        plsc.subcore_barrier()
        ob = jnp.minimum(zb, n_dst - per_rows)
        pltpu.sync_copy(acc.at[pl.ds(ob, per_rows)],
                        out_h.at[c, pl.ds(ob, per_rows)])

    return k(rowm, colm, vals, x, zeros)


def _pad_edges(r, c, v, e_pad, n_dst, n_src):
    pad = e_pad - r.shape[0]
    spread = jnp.arange(pad, dtype=I32)
    r = jnp.concatenate([r.astype(I32), spread % n_dst])
    c = jnp.concatenate([c.astype(I32), spread % n_src])
    v = jnp.concatenate([v, jnp.zeros((pad,), F32)])
    return r, c, v


def kernel(image_x, text_x, cat_x, A_ii_idx, A_ii_val, A_oi_row, A_oi_col,
           A_oi_val, A_uo_row, A_uo_col, A_uo_val, params):
    p = params
    zeros = jnp.zeros((_HALFP, _D), F32)

    x = _proj_fuse(image_x, text_x, cat_x, p)

    ii_groups, ii_qsuper = 26, 2
    ii_gran = 16 * ii_groups * ii_qsuper * _SB * _CH
    e_pad = ii_gran * pl.cdiv(A_ii_val.shape[0], ii_gran)
    r, c, v = _pad_edges(A_ii_idx[0], A_ii_idx[1], A_ii_val, e_pad, _NI, _NI)
    x_prop = _spmm_ii(r, c, v, x, zeros, ii_groups, ii_qsuper)

    item_emb = _item_update(x, x_prop, p)

    e_pad = 32 * _SB * _CH * pl.cdiv(A_oi_val.shape[0], 32 * _SB * _CH)
    r, c, v = _pad_edges(A_oi_row, A_oi_col, A_oi_val, e_pad, _NO, _NI)
    agg = _spmm_part(r, c, v, item_emb, zeros, _NO)
    outfit_emb = _node_update(agg[0], agg[1], p['outfit_base'], p, 'ou')

    e_pad = 32 * _SB * _CH * pl.cdiv(A_uo_val.shape[0], 32 * _SB * _CH)
    r, c, v = _pad_edges(A_uo_row, A_uo_col, A_uo_val, e_pad, _NU, _NO)
    agg2 = _spmm_part(r, c, v, outfit_emb, zeros, _NU)
    user_emb = _node_update(agg2[0], agg2[1], p['user_base'], p, 'uu')

    return (user_emb, outfit_emb, item_emb)


# speedup vs baseline: 1.1149x; 1.0381x over previous
import functools

import jax
import jax.numpy as jnp
from jax import lax
from jax.experimental import pallas as pl
from jax.experimental.pallas import tpu as pltpu
from jax.experimental.pallas import tpu_sc as plsc

F32 = jnp.float32
I32 = jnp.int32

_D = 64
_L = 16
_CH = 128
_NI = 50000
_NO = 10000
_NU = 10000

_HALF = 25000
_HALFP = 25088
_ZROWS_II = _HALFP // 16

_BLK = 2048


def _l2n(x):
    n = jnp.sqrt(jnp.sum(x * x, axis=-1, keepdims=True))
    return x / jnp.maximum(n, 1e-12)


def _ln_relu(h, g, be):
    mu = jnp.mean(h, axis=-1, keepdims=True)
    var = jnp.mean((h - mu) ** 2, axis=-1, keepdims=True)
    h = (h - mu) / jnp.sqrt(var + 1e-5) * g + be
    return jnp.maximum(h, 0.0)


def _dot(a, b):
    return jnp.dot(a, b, preferred_element_type=F32)


def _fuse_body(img, txt, cat, wi, bi, wt, bt, wc, bc,
               w1i, w1t, w1c, b1, g, be, w2, b2, o):
    xi = _l2n(_dot(img[...], wi[...]) + bi[...])
    xt = _l2n(_dot(txt[...], wt[...]) + bt[...])
    xc = _l2n(_dot(cat[...], wc[...]) + bc[...])
    h = (_dot(xi, w1i[...]) + _dot(xt, w1t[...]) + _dot(xc, w1c[...])
         + b1[...])
    h = _ln_relu(h, g[...], be[...])
    o[...] = _l2n(_dot(h, w2[...]) + b2[...])


def _proj_fuse(image_x, text_x, cat_x, p):
    grid = (pl.cdiv(_NI, _BLK),)

    def row_spec(cols):
        return pl.BlockSpec((_BLK, cols), lambda i: (i, 0))

    def full_spec(shape):
        return pl.BlockSpec(shape, lambda i: (0,) * len(shape))

    w1 = p['fuse_W1']
    args = (image_x, text_x, cat_x,
            p['img_W'], p['img_b'].reshape(1, _D),
            p['txt_W'], p['txt_b'].reshape(1, _D),
            p['cat_W'], p['cat_b'].reshape(1, _D),
            w1[:_D], w1[_D:2 * _D], w1[2 * _D:],
            p['fuse_b1'].reshape(1, _D), p['fuse_g'].reshape(1, _D),
            p['fuse_be'].reshape(1, _D), p['fuse_W2'],
            p['fuse_b2'].reshape(1, _D))
    in_specs = [row_spec(512), row_spec(384), row_spec(128)] + [
        full_spec(a.shape) for a in args[3:]]
    return pl.pallas_call(
        _fuse_body,
        grid=grid,
        in_specs=in_specs,
        out_specs=pl.BlockSpec((_BLK, _D), lambda i: (i, 0)),
        out_shape=jax.ShapeDtypeStruct((_NI, _D), F32),
    )(*args)


def _item_body(x, xp, w1, b1, g, be, w2, b2, o):
    h = _ln_relu(_dot(xp[...], w1[...]) + b1[...], g[...], be[...])
    o[...] = _l2n(x[...] + _dot(h, w2[...]) + b2[...])


def _item_update(x, x_prop, p):
    grid = (pl.cdiv(_NI, _BLK),)
    args = (x, x_prop, p['iu_W1'], p['iu_b1'].reshape(1, _D),
            p['iu_g'].reshape(1, _D), p['iu_be'].reshape(1, _D),
            p['iu_W2'], p['iu_b2'].reshape(1, _D))
    in_specs = [pl.BlockSpec((_BLK, _D), lambda i: (i, 0))] * 2 + [
        pl.BlockSpec(a.shape, lambda i: (0, 0)) for a in args[2:]]
    return pl.pallas_call(
        _item_body,
        grid=grid,
        in_specs=in_specs,
        out_specs=pl.BlockSpec((_BLK, _D), lambda i: (i, 0)),
        out_shape=jax.ShapeDtypeStruct((_NI, _D), F32),
    )(*args)


def _node_body(a0, a1, base, w1, b1, g, be, w2, b2, o):
    agg = a0[...] + a1[...]
    h = _ln_relu(_dot(agg, w1[...]) + b1[...], g[...], be[...])
    o[...] = _l2n(_l2n(base[...]) + _dot(h, w2[...]) + b2[...])


def _node_update(a0, a1, base, p, name):
    n = base.shape[0]
    grid = (pl.cdiv(n, _BLK),)
    args = (a0, a1, base, p[name + '_W1'], p[name + '_b1'].reshape(1, _D),
            p[name + '_g'].reshape(1, _D), p[name + '_be'].reshape(1, _D),
            p[name + '_W2'], p[name + '_b2'].reshape(1, _D))
    in_specs = [pl.BlockSpec((_BLK, _D), lambda i: (i, 0))] * 3 + [
        pl.BlockSpec(a.shape, lambda i: (0, 0)) for a in args[3:]]
    return pl.pallas_call(
        _node_body,
        grid=grid,
        in_specs=in_specs,
        out_specs=pl.BlockSpec((_BLK, _D), lambda i: (i, 0)),
        out_shape=jax.ShapeDtypeStruct((n, _D), F32),
    )(*args)


_SB = 8


def _scale_chunk(valq, buf, voff):
    @plsc.parallel_loop(0, _CH, unroll=8)
    def edge(i):
        vb = jnp.full((_L,), valq[pl.ds(voff + i, _L)][0], F32)
        for j in range(_D // _L):
            sl = pl.ds(j * _L, _L)
            buf[i, sl] = buf[i, sl] * vb


def _process_super(x_h, acc, colq, idxq, valq, gbufs, gsems, ssems,
                   base_row, transform):
    transform(base_row)
    cps = [None] * 2
    scs = [None] * 2
    cps[0] = pltpu.async_copy(x_h.at[colq.at[base_row]], gbufs[0], gsems[0])
    for b in range(_SB):
        if b + 1 < _SB:
            k = (b + 1) % 2
            if b >= 1:
                scs[k].wait()
            cps[k] = pltpu.async_copy(x_h.at[colq.at[base_row + b + 1]],
                                      gbufs[k], gsems[k])
        cps[b % 2].wait()
        _scale_chunk(valq, gbufs[b % 2], (base_row + b) * _CH)
        scs[b % 2] = pltpu.async_copy(gbufs[b % 2],
                                      acc.at[idxq.at[base_row + b]],
                                      ssems[b % 2], add=True)
    scs[(_SB - 2) % 2].wait()
    scs[(_SB - 1) % 2].wait()


def _process_group(x_h, acc, colq, idxq, valq, gbufs, gsems, ssems,
                   n_super, transform):
    def sup(u, carry):
        _process_super(x_h, acc, colq, idxq, valq, gbufs, gsems, ssems,
                       u * _SB, transform)
        return carry
    lax.fori_loop(0, n_super, sup, 0)


def _stage(colm, rowm, valf, colq, idxq, valq, sems, row0, q_rows):
    pltpu.async_copy(colm.at[pl.ds(row0, q_rows)], colq, sems[0])
    pltpu.async_copy(rowm.at[pl.ds(row0, q_rows)], idxq, sems[1])
    pltpu.async_copy(valf.at[pl.ds(row0 * _CH, q_rows * _CH)],
                     valq.at[pl.ds(0, q_rows * _CH)], sems[2])


def _stage_wait(colm, rowm, valf, colq, idxq, valq, sems, row0, q_rows):
    pltpu.make_async_copy(colm.at[pl.ds(row0, q_rows)], colq, sems[0]).wait()
    pltpu.make_async_copy(rowm.at[pl.ds(row0, q_rows)], idxq, sems[1]).wait()
    pltpu.make_async_copy(valf.at[pl.ds(row0 * _CH, q_rows * _CH)],
                          valq.at[pl.ds(0, q_rows * _CH)], sems[2]).wait()


def _sc_scratch(n_acc_rows, q_rows, double):
    n_sets = 2 if double else 1
    sc = []
    for _ in range(n_sets):
        sc += [
            pltpu.VMEM((q_rows, _CH), I32),
            pltpu.VMEM((q_rows, _CH), I32),
            pltpu.VMEM((q_rows * _CH + _L,), F32),
            (pltpu.SemaphoreType.DMA,) * 3,
        ]
    sc += [
        tuple(pltpu.VMEM((_CH, _D), F32) for _ in range(2)),
        (pltpu.SemaphoreType.DMA,) * 2,
        (pltpu.SemaphoreType.DMA,) * 2,
        pltpu.VMEM_SHARED((n_acc_rows, _D), F32),
    ]
    return sc


def _spmm_ii(rows, cols, vals, x, zeros, n_groups, q_super):
    e_pad = rows.shape[0]
    per_tile_rows = e_pad // _CH // 16
    q_rows = q_super * _SB
    n_pairs = n_groups // 2
    rowm = rows.reshape(-1, _CH)
    colm = cols.reshape(-1, _CH)
    mesh = plsc.VectorSubcoreMesh(core_axis_name="c", subcore_axis_name="s")

    @functools.partial(
        pl.kernel,
        out_type=jax.ShapeDtypeStruct((_NI, _D), F32),
        mesh=mesh,
        scratch_types=_sc_scratch(_HALFP, q_rows, True),
        compiler_params=pltpu.CompilerParams(use_tc_tiling_on_sc=False),
    )
    def k(row_h, col_h, val_h, x_h, z_h, out_h,
          col_a, idx_a, val_a, st_a, col_b, idx_b, val_b, st_b,
          gbufs, gsems, ssems, acc):
        c = lax.axis_index("c")
        s = lax.axis_index("s")
        zb = s * _ZROWS_II
        pltpu.sync_copy(z_h.at[pl.ds(zb, _ZROWS_II)],
                        acc.at[pl.ds(zb, _ZROWS_II)])
        plsc.subcore_barrier()

        def make_transform(idxq, valq):
            def transform(base_row):
                for b in range(_SB):
                    for g in range(_CH // _L):
                        sl = pl.ds(g * _L, _L)
                        r = idxq[base_row + b, sl]
                        hi = r >= _HALF
                        idxq[base_row + b, sl] = jnp.where(hi, r - _HALF, r)
                        side = jnp.where(hi, jnp.ones((_L,), I32),
                                         jnp.zeros((_L,), I32))
                        vsl = pl.ds((base_row + b) * _CH + g * _L, _L)
                        valq[vsl] = jnp.where(side == c, valq[vsl],
                                              jnp.zeros((_L,), F32))
            return transform

        tile_row0 = s * per_tile_rows
        set_a = (col_a, idx_a, val_a, st_a)
        set_b = (col_b, idx_b, val_b, st_b)

        def stage_for(st, row0):
            _stage(col_h, row_h, val_h, st[0], st[1], st[2], st[3],
                   row0, q_rows)

        def wait_for(st, row0):
            _stage_wait(col_h, row_h, val_h, st[0], st[1], st[2], st[3],
                        row0, q_rows)

        def process(st):
            _process_group(x_h, acc, st[0], st[1], st[2], gbufs, gsems,
                           ssems, q_super, make_transform(st[1], st[2]))

        stage_for(set_a, tile_row0)

        def pair(q, carry):
            ra = tile_row0 + (2 * q) * q_rows
            rb = ra + q_rows
            wait_for(set_a, ra)
            stage_for(set_b, rb)
            process(set_a)

            @pl.when(q + 1 < n_pairs)
            def _():
                stage_for(set_a, ra + 2 * q_rows)

            wait_for(set_b, rb)
            process(set_b)
            return carry

        lax.fori_loop(0, n_pairs, pair, 0)
        plsc.subcore_barrier()
        ob = jnp.minimum(s * _ZROWS_II, _HALF - _ZROWS_II)
        pltpu.sync_copy(acc.at[pl.ds(ob, _ZROWS_II)],
                        out_h.at[pl.ds(c * _HALF + ob, _ZROWS_II)])

    return k(rowm, colm, vals, x, zeros)


def _spmm_part(rows, cols, vals, x, zeros, n_dst):
    n_pad = 128 * pl.cdiv(n_dst, 128)
    per_rows = n_pad // 16
    e_pad = rows.shape[0]
    per_tile_rows = e_pad // _CH // 32
    q_super = per_tile_rows // _SB
    q_rows = per_tile_rows
    rowm = rows.reshape(-1, _CH)
    colm = cols.reshape(-1, _CH)
    zeros = zeros[:n_pad]
    mesh = plsc.VectorSubcoreMesh(core_axis_name="c", subcore_axis_name="s")

    @functools.partial(
        pl.kernel,
        out_type=jax.ShapeDtypeStruct((2, n_dst, _D), F32),
        mesh=mesh,
        scratch_types=_sc_scratch(n_pad, q_rows, False),
        compiler_params=pltpu.CompilerParams(use_tc_tiling_on_sc=False),
    )
    def k(row_h, col_h, val_h, x_h, z_h, out_h,
          col_a, idx_a, val_a, st_a, gbufs, gsems, ssems, acc):
        c = lax.axis_index("c")
        s = lax.axis_index("s")
        zb = s * per_rows
        pltpu.sync_copy(z_h.at[pl.ds(zb, per_rows)],
                        acc.at[pl.ds(zb, per_rows)])
        plsc.subcore_barrier()

        row0 = (s * 2 + c) * per_tile_rows
        _stage(col_h, row_h, val_h, col_a, idx_a, val_a, st_a, row0, q_rows)
        _stage_wait(col_h, row_h, val_h, col_a, idx_a, val_a, st_a,
                    row0, q_rows)
        _process_group(x_h, acc, col_a, idx_a, val_a, gbufs, gsems, ssems,
                       q_super, lambda base_row: None)
        plsc.subcore_barrier()
        ob = jnp.minimum(zb, n_dst - per_rows)
        pltpu.sync_copy(acc.at[pl.ds(ob, per_rows)],
                        out_h.at[c, pl.ds(ob, per_rows)])

    return k(rowm, colm, vals, x, zeros)


def _pad_edges(r, c, v, e_pad, n_dst, n_src):
    pad = e_pad - r.shape[0]
    spread = jnp.arange(pad, dtype=I32)
    r = jnp.concatenate([r.astype(I32), spread % n_dst])
    c = jnp.concatenate([c.astype(I32), spread % n_src])
    v = jnp.concatenate([v, jnp.zeros((pad,), F32)])
    return r, c, v


def kernel(image_x, text_x, cat_x, A_ii_idx, A_ii_val, A_oi_row, A_oi_col,
           A_oi_val, A_uo_row, A_uo_col, A_uo_val, params):
    p = params
    zeros = jnp.zeros((_HALFP, _D), F32)

    x = _proj_fuse(image_x, text_x, cat_x, p)

    ii_groups, ii_qsuper = 26, 2
    ii_gran = 16 * ii_groups * ii_qsuper * _SB * _CH
    e_pad = ii_gran * pl.cdiv(A_ii_val.shape[0], ii_gran)
    r, c, v = _pad_edges(A_ii_idx[0], A_ii_idx[1], A_ii_val, e_pad, _NI, _NI)
    x_prop = _spmm_ii(r, c, v, x, zeros, ii_groups, ii_qsuper)

    item_emb = _item_update(x, x_prop, p)

    e_pad = 32 * _SB * _CH * pl.cdiv(A_oi_val.shape[0], 32 * _SB * _CH)
    r, c, v = _pad_edges(A_oi_row, A_oi_col, A_oi_val, e_pad, _NO, _NI)
    agg = _spmm_part(r, c, v, item_emb, zeros, _NO)
    outfit_emb = _node_update(agg[0], agg[1], p['outfit_base'], p, 'ou')

    e_pad = 32 * _SB * _CH * pl.cdiv(A_uo_val.shape[0], 32 * _SB * _CH)
    r, c, v = _pad_edges(A_uo_row, A_uo_col, A_uo_val, e_pad, _NU, _NO)
    agg2 = _spmm_part(r, c, v, outfit_emb, zeros, _NU)
    user_emb = _node_update(agg2[0], agg2[1], p['user_base'], p, 'uu')

    return (user_emb, outfit_emb, item_emb)


# speedup vs baseline: 1.1196x; 1.0042x over previous
import functools

import jax
import jax.numpy as jnp
from jax import lax
from jax.experimental import pallas as pl
from jax.experimental.pallas import tpu as pltpu
from jax.experimental.pallas import tpu_sc as plsc

F32 = jnp.float32
I32 = jnp.int32

_D = 64
_L = 16
_CH = 128
_NI = 50000
_NO = 10000
_NU = 10000

_HALF = 25000
_HALFP = 25088
_ZROWS_II = _HALFP // 16

_BLK = 4096


def _l2n(x):
    n = jnp.sqrt(jnp.sum(x * x, axis=-1, keepdims=True))
    return x / jnp.maximum(n, 1e-12)


def _ln_relu(h, g, be):
    mu = jnp.mean(h, axis=-1, keepdims=True)
    var = jnp.mean((h - mu) ** 2, axis=-1, keepdims=True)
    h = (h - mu) / jnp.sqrt(var + 1e-5) * g + be
    return jnp.maximum(h, 0.0)


def _dot(a, b):
    return jnp.dot(a, b, preferred_element_type=F32)


def _fuse_body(img, txt, cat, wi, bi, wt, bt, wc, bc,
               w1i, w1t, w1c, b1, g, be, w2, b2, o):
    xi = _l2n(_dot(img[...], wi[...]) + bi[...])
    xt = _l2n(_dot(txt[...], wt[...]) + bt[...])
    xc = _l2n(_dot(cat[...], wc[...]) + bc[...])
    h = (_dot(xi, w1i[...]) + _dot(xt, w1t[...]) + _dot(xc, w1c[...])
         + b1[...])
    h = _ln_relu(h, g[...], be[...])
    o[...] = _l2n(_dot(h, w2[...]) + b2[...])


def _proj_fuse(image_x, text_x, cat_x, p):
    grid = (pl.cdiv(_NI, _BLK),)

    def row_spec(cols):
        return pl.BlockSpec((_BLK, cols), lambda i: (i, 0))

    def full_spec(shape):
        return pl.BlockSpec(shape, lambda i: (0,) * len(shape))

    w1 = p['fuse_W1']
    args = (image_x, text_x, cat_x,
            p['img_W'], p['img_b'].reshape(1, _D),
            p['txt_W'], p['txt_b'].reshape(1, _D),
            p['cat_W'], p['cat_b'].reshape(1, _D),
            w1[:_D], w1[_D:2 * _D], w1[2 * _D:],
            p['fuse_b1'].reshape(1, _D), p['fuse_g'].reshape(1, _D),
            p['fuse_be'].reshape(1, _D), p['fuse_W2'],
            p['fuse_b2'].reshape(1, _D))
    in_specs = [row_spec(512), row_spec(384), row_spec(128)] + [
        full_spec(a.shape) for a in args[3:]]
    return pl.pallas_call(
        _fuse_body,
        grid=grid,
        in_specs=in_specs,
        out_specs=pl.BlockSpec((_BLK, _D), lambda i: (i, 0)),
        out_shape=jax.ShapeDtypeStruct((_NI, _D), F32),
    )(*args)


def _item_body(x, xp, w1, b1, g, be, w2, b2, o):
    h = _ln_relu(_dot(xp[...], w1[...]) + b1[...], g[...], be[...])
    o[...] = _l2n(x[...] + _dot(h, w2[...]) + b2[...])


def _item_update(x, x_prop, p):
    grid = (pl.cdiv(_NI, _BLK),)
    args = (x, x_prop, p['iu_W1'], p['iu_b1'].reshape(1, _D),
            p['iu_g'].reshape(1, _D), p['iu_be'].reshape(1, _D),
            p['iu_W2'], p['iu_b2'].reshape(1, _D))
    in_specs = [pl.BlockSpec((_BLK, _D), lambda i: (i, 0))] * 2 + [
        pl.BlockSpec(a.shape, lambda i: (0, 0)) for a in args[2:]]
    return pl.pallas_call(
        _item_body,
        grid=grid,
        in_specs=in_specs,
        out_specs=pl.BlockSpec((_BLK, _D), lambda i: (i, 0)),
        out_shape=jax.ShapeDtypeStruct((_NI, _D), F32),
    )(*args)


def _node_body(a0, a1, base, w1, b1, g, be, w2, b2, o):
    agg = a0[...] + a1[...]
    h = _ln_relu(_dot(agg, w1[...]) + b1[...], g[...], be[...])
    o[...] = _l2n(_l2n(base[...]) + _dot(h, w2[...]) + b2[...])


def _node_update(a0, a1, base, p, name):
    n = base.shape[0]
    grid = (pl.cdiv(n, _BLK),)
    args = (a0, a1, base, p[name + '_W1'], p[name + '_b1'].reshape(1, _D),
            p[name + '_g'].reshape(1, _D), p[name + '_be'].reshape(1, _D),
            p[name + '_W2'], p[name + '_b2'].reshape(1, _D))
    in_specs = [pl.BlockSpec((_BLK, _D), lambda i: (i, 0))] * 3 + [
        pl.BlockSpec(a.shape, lambda i: (0, 0)) for a in args[3:]]
    return pl.pallas_call(
        _node_body,
        grid=grid,
        in_specs=in_specs,
        out_specs=pl.BlockSpec((_BLK, _D), lambda i: (i, 0)),
        out_shape=jax.ShapeDtypeStruct((n, _D), F32),
    )(*args)


_SB = 8


def _scale_chunk(valq, buf, voff):
    @plsc.parallel_loop(0, _CH, unroll=8)
    def edge(i):
        vb = jnp.full((_L,), valq[pl.ds(voff + i, _L)][0], F32)
        for j in range(_D // _L):
            sl = pl.ds(j * _L, _L)
            buf[i, sl] = buf[i, sl] * vb


def _process_super(x_h, acc, colq, idxq, valq, gbufs, gsems, ssems,
                   base_row, transform):
    transform(base_row)
    cps = [None] * 2
    scs = [None] * 2
    cps[0] = pltpu.async_copy(x_h.at[colq.at[base_row]], gbufs[0], gsems[0])
    for b in range(_SB):
        if b + 1 < _SB:
            k = (b + 1) % 2
            if b >= 1:
                scs[k].wait()
            cps[k] = pltpu.async_copy(x_h.at[colq.at[base_row + b + 1]],
                                      gbufs[k], gsems[k])
        cps[b % 2].wait()
        _scale_chunk(valq, gbufs[b % 2], (base_row + b) * _CH)
        scs[b % 2] = pltpu.async_copy(gbufs[b % 2],
                                      acc.at[idxq.at[base_row + b]],
                                      ssems[b % 2], add=True)
    scs[(_SB - 2) % 2].wait()
    scs[(_SB - 1) % 2].wait()


def _process_group(x_h, acc, colq, idxq, valq, gbufs, gsems, ssems,
                   n_super, transform):
    def sup(u, carry):
        _process_super(x_h, acc, colq, idxq, valq, gbufs, gsems, ssems,
                       u * _SB, transform)
        return carry
    lax.fori_loop(0, n_super, sup, 0)


def _stage(colm, rowm, valf, colq, idxq, valq, sems, row0, q_rows):
    pltpu.async_copy(colm.at[pl.ds(row0, q_rows)], colq, sems[0])
    pltpu.async_copy(rowm.at[pl.ds(row0, q_rows)], idxq, sems[1])
    pltpu.async_copy(valf.at[pl.ds(row0 * _CH, q_rows * _CH)],
                     valq.at[pl.ds(0, q_rows * _CH)], sems[2])


def _stage_wait(colm, rowm, valf, colq, idxq, valq, sems, row0, q_rows):
    pltpu.make_async_copy(colm.at[pl.ds(row0, q_rows)], colq, sems[0]).wait()
    pltpu.make_async_copy(rowm.at[pl.ds(row0, q_rows)], idxq, sems[1]).wait()
    pltpu.make_async_copy(valf.at[pl.ds(row0 * _CH, q_rows * _CH)],
                          valq.at[pl.ds(0, q_rows * _CH)], sems[2]).wait()


def _sc_scratch(n_acc_rows, q_rows, double):
    n_sets = 2 if double else 1
    sc = []
    for _ in range(n_sets):
        sc += [
            pltpu.VMEM((q_rows, _CH), I32),
            pltpu.VMEM((q_rows, _CH), I32),
            pltpu.VMEM((q_rows * _CH + _L,), F32),
            (pltpu.SemaphoreType.DMA,) * 3,
        ]
    sc += [
        tuple(pltpu.VMEM((_CH, _D), F32) for _ in range(2)),
        (pltpu.SemaphoreType.DMA,) * 2,
        (pltpu.SemaphoreType.DMA,) * 2,
        pltpu.VMEM_SHARED((n_acc_rows, _D), F32),
    ]
    return sc


def _spmm_ii(rows, cols, vals, x, zeros, n_groups, q_super):
    e_pad = rows.shape[0]
    per_tile_rows = e_pad // _CH // 16
    q_rows = q_super * _SB
    n_pairs = n_groups // 2
    rowm = rows.reshape(-1, _CH)
    colm = cols.reshape(-1, _CH)
    mesh = plsc.VectorSubcoreMesh(core_axis_name="c", subcore_axis_name="s")

    @functools.partial(
        pl.kernel,
        out_type=jax.ShapeDtypeStruct((_NI, _D), F32),
        mesh=mesh,
        scratch_types=_sc_scratch(_HALFP, q_rows, True),
        compiler_params=pltpu.CompilerParams(use_tc_tiling_on_sc=False),
    )
    def k(row_h, col_h, val_h, x_h, z_h, out_h,
          col_a, idx_a, val_a, st_a, col_b, idx_b, val_b, st_b,
          gbufs, gsems, ssems, acc):
        c = lax.axis_index("c")
        s = lax.axis_index("s")
        zb = s * _ZROWS_II
        pltpu.sync_copy(z_h.at[pl.ds(zb, _ZROWS_II)],
                        acc.at[pl.ds(zb, _ZROWS_II)])
        plsc.subcore_barrier()

        def make_transform(idxq, valq):
            def transform(base_row):
                for b in range(_SB):
                    for g in range(_CH // _L):
                        sl = pl.ds(g * _L, _L)
                        r = idxq[base_row + b, sl]
                        hi = r >= _HALF
                        idxq[base_row + b, sl] = jnp.where(hi, r - _HALF, r)
                        side = jnp.where(hi, jnp.ones((_L,), I32),
                                         jnp.zeros((_L,), I32))
                        vsl = pl.ds((base_row + b) * _CH + g * _L, _L)
                        valq[vsl] = jnp.where(side == c, valq[vsl],
                                              jnp.zeros((_L,), F32))
            return transform

        tile_row0 = s * per_tile_rows
        set_a = (col_a, idx_a, val_a, st_a)
        set_b = (col_b, idx_b, val_b, st_b)

        def stage_for(st, row0):
            _stage(col_h, row_h, val_h, st[0], st[1], st[2], st[3],
                   row0, q_rows)

        def wait_for(st, row0):
            _stage_wait(col_h, row_h, val_h, st[0], st[1], st[2], st[3],
                        row0, q_rows)

        def process(st):
            _process_group(x_h, acc, st[0], st[1], st[2], gbufs, gsems,
                           ssems, q_super, make_transform(st[1], st[2]))

        stage_for(set_a, tile_row0)

        def pair(q, carry):
            ra = tile_row0 + (2 * q) * q_rows
            rb = ra + q_rows
            wait_for(set_a, ra)
            stage_for(set_b, rb)
            process(set_a)

            @pl.when(q + 1 < n_pairs)
            def _():
                stage_for(set_a, ra + 2 * q_rows)

            wait_for(set_b, rb)
            process(set_b)
            return carry

        lax.fori_loop(0, n_pairs, pair, 0)
        plsc.subcore_barrier()
        ob = jnp.minimum(s * _ZROWS_II, _HALF - _ZROWS_II)
        pltpu.sync_copy(acc.at[pl.ds(ob, _ZROWS_II)],
                        out_h.at[pl.ds(c * _HALF + ob, _ZROWS_II)])

    return k(rowm, colm, vals, x, zeros)


def _spmm_part(rows, cols, vals, x, zeros, n_dst):
    n_pad = 128 * pl.cdiv(n_dst, 128)
    per_rows = n_pad // 16
    e_pad = rows.shape[0]
    per_tile_rows = e_pad // _CH // 32
    q_super = per_tile_rows // _SB
    q_rows = per_tile_rows
    rowm = rows.reshape(-1, _CH)
    colm = cols.reshape(-1, _CH)
    zeros = zeros[:n_pad]
    mesh = plsc.VectorSubcoreMesh(core_axis_name="c", subcore_axis_name="s")

    @functools.partial(
        pl.kernel,
        out_type=jax.ShapeDtypeStruct((2, n_dst, _D), F32),
        mesh=mesh,
        scratch_types=_sc_scratch(n_pad, q_rows, False),
        compiler_params=pltpu.CompilerParams(use_tc_tiling_on_sc=False),
    )
    def k(row_h, col_h, val_h, x_h, z_h, out_h,
          col_a, idx_a, val_a, st_a, gbufs, gsems, ssems, acc):
        c = lax.axis_index("c")
        s = lax.axis_index("s")
        zb = s * per_rows
        pltpu.sync_copy(z_h.at[pl.ds(zb, per_rows)],
                        acc.at[pl.ds(zb, per_rows)])
        plsc.subcore_barrier()

        row0 = (s * 2 + c) * per_tile_rows
        _stage(col_h, row_h, val_h, col_a, idx_a, val_a, st_a, row0, q_rows)
        _stage_wait(col_h, row_h, val_h, col_a, idx_a, val_a, st_a,
                    row0, q_rows)
        _process_group(x_h, acc, col_a, idx_a, val_a, gbufs, gsems, ssems,
                       q_super, lambda base_row: None)
        plsc.subcore_barrier()
        ob = jnp.minimum(zb, n_dst - per_rows)
        pltpu.sync_copy(acc.at[pl.ds(ob, per_rows)],
                        out_h.at[c, pl.ds(ob, per_rows)])

    return k(rowm, colm, vals, x, zeros)


def _pad_edges(r, c, v, e_pad, n_dst, n_src):
    pad = e_pad - r.shape[0]
    spread = jnp.arange(pad, dtype=I32)
    r = jnp.concatenate([r.astype(I32), spread % n_dst])
    c = jnp.concatenate([c.astype(I32), spread % n_src])
    v = jnp.concatenate([v, jnp.zeros((pad,), F32)])
    return r, c, v


def kernel(image_x, text_x, cat_x, A_ii_idx, A_ii_val, A_oi_row, A_oi_col,
           A_oi_val, A_uo_row, A_uo_col, A_uo_val, params):
    p = params
    zeros = jnp.zeros((_HALFP, _D), F32)

    x = _proj_fuse(image_x, text_x, cat_x, p)

    ii_groups, ii_qsuper = 26, 2
    ii_gran = 16 * ii_groups * ii_qsuper * _SB * _CH
    e_pad = ii_gran * pl.cdiv(A_ii_val.shape[0], ii_gran)
    r, c, v = _pad_edges(A_ii_idx[0], A_ii_idx[1], A_ii_val, e_pad, _NI, _NI)
    x_prop = _spmm_ii(r, c, v, x, zeros, ii_groups, ii_qsuper)

    item_emb = _item_update(x, x_prop, p)

    e_pad = 32 * _SB * _CH * pl.cdiv(A_oi_val.shape[0], 32 * _SB * _CH)
    r, c, v = _pad_edges(A_oi_row, A_oi_col, A_oi_val, e_pad, _NO, _NI)
    agg = _spmm_part(r, c, v, item_emb, zeros, _NO)
    outfit_emb = _node_update(agg[0], agg[1], p['outfit_base'], p, 'ou')

    e_pad = 32 * _SB * _CH * pl.cdiv(A_uo_val.shape[0], 32 * _SB * _CH)
    r, c, v = _pad_edges(A_uo_row, A_uo_col, A_uo_val, e_pad, _NU, _NO)
    agg2 = _spmm_part(r, c, v, outfit_emb, zeros, _NU)
    user_emb = _node_update(agg2[0], agg2[1], p['user_base'], p, 'uu')

    return (user_emb, outfit_emb, item_emb)
